# baseline (speedup 1.0000x reference)
import functools

import jax
import jax.numpy as jnp
from jax import lax
from jax.experimental import pallas as pl
from jax.experimental.pallas import tpu as pltpu
from jax.experimental.pallas import tpu_sc as plsc

_BATCH = 16384
_FDIM = 2048
_MASK = 256

_NC = 2
_NS = 16
_NW = _NC * _NS
_ROWS_PER_W = _BATCH // _NW
_CH = 8
_CHE = _CH * _FDIM
_NBUF = 4
_NCHUNK = _ROWS_PER_W // _CH
_NGRP = _NCHUNK // _NBUF
_SIDX = _CH * _MASK


def _sc_kernel(x_hbm, idx_hbm, out_hbm, idx_v, sidx_v, b0, b1, b2, b3,
               si0, si1, si2, si3, so0, so1, so2, so3):
    bufs = (b0, b1, b2, b3)
    sin = (si0, si1, si2, si3)
    sout = (so0, so1, so2, so3)

    wid = lax.axis_index("c") * _NS + lax.axis_index("s")
    elem0 = wid * (_ROWS_PER_W * _FDIM)

    pltpu.sync_copy(idx_hbm, idx_v)
    zeros16 = jnp.zeros((16,), jnp.float32)

    for r in range(_CH):
        for k in range(_MASK // 16):
            sidx_v[pl.ds(r * _MASK + k * 16, 16)] = (
                idx_v[pl.ds(k * 16, 16)] + r * _FDIM
            )

    def start_in(c, b):
        pltpu.make_async_copy(
            x_hbm.at[pl.ds(elem0 + c * _CHE, _CHE)], bufs[b], sin[b]
        ).start()

    def wait_in(b):
        pltpu.make_async_copy(
            x_hbm.at[pl.ds(elem0, _CHE)], bufs[b], sin[b]
        ).wait()

    def start_out(c, b):
        pltpu.make_async_copy(
            bufs[b], out_hbm.at[pl.ds(elem0 + c * _CHE, _CHE)], sout[b]
        ).start()

    def wait_out(b):
        pltpu.make_async_copy(
            bufs[b], out_hbm.at[pl.ds(elem0, _CHE)], sout[b]
        ).wait()

    def scatter_zeros(b):
        buf = bufs[b]
        for t in range(_SIDX // 16):
            plsc.store_scatter(buf, [sidx_v[pl.ds(t * 16, 16)]], zeros16)

    for b in range(_NBUF):
        start_in(b, b)

    def group(i, carry):
        for b in range(_NBUF):
            c = i * _NBUF + b
            wait_in(b)
            scatter_zeros(b)
            start_out(c, b)
            wait_out(b)
            start_in(c + _NBUF, b)
        return carry

    lax.fori_loop(0, _NGRP - 1, group, 0)

    for b in range(_NBUF):
        c = (_NGRP - 1) * _NBUF + b
        wait_in(b)
        scatter_zeros(b)
        start_out(c, b)
    for b in range(_NBUF):
        wait_out(b)


def kernel(x, mask_indices):
    mesh = plsc.VectorSubcoreMesh(core_axis_name="c", subcore_axis_name="s")
    f = functools.partial(
        pl.kernel,
        mesh=mesh,
        out_type=jax.ShapeDtypeStruct((_BATCH * _FDIM,), jnp.float32),
        scratch_types=[
            pltpu.VMEM((_MASK,), jnp.int32),
            pltpu.VMEM((_SIDX,), jnp.int32),
        ] + [pltpu.VMEM((_CHE,), jnp.float32) for _ in range(_NBUF)]
        + [pltpu.SemaphoreType.DMA for _ in range(2 * _NBUF)],
        compiler_params=pltpu.CompilerParams(needs_layout_passes=False),
    )(_sc_kernel)
    out = f(x.reshape(-1), mask_indices)
    return out.reshape(_BATCH, _FDIM)
